# Initial kernel scaffold; baseline (speedup 1.0000x reference)
import jax
import jax.numpy as jnp
from jax.experimental import pallas as pl


def kernel(X, h1_node_idx, h1_edge_idx, h1_DV_inv_sqrt, h1_DE_inv, h2_node_idx, h2_edge_idx, h2_DV_inv_sqrt, h2_DE_inv, W1, b1, W2, b2, g1, beta1, g2, beta2, Wa, ba):
    raise NotImplementedError("write your pallas kernel here")


# speedup vs baseline: 2.3084x; 2.3084x over previous
import functools

import jax
import jax.numpy as jnp
from jax import lax
from jax.experimental import pallas as pl
from jax.experimental.pallas import tpu as pltpu
from jax.experimental.pallas import tpu_sc as plsc

N = 10000
D = 256
EH = 5000
NNZ = 160000
HD = 128

NT = 16
C = 80
NNZ_T = NNZ // NT
N_ITER = NNZ_T // C
E_T = 320
EHP = NT * E_T
NODE_T = 632
NP = NT * NODE_T
ZR = 40
BN = 1000


def _zero_vmem(zb, nrows):
    def _zero_row(r, _):
        for v in range(HD // 16):
            zb[r, pl.ds(v * 16, 16)] = jnp.zeros((16,), jnp.float32)
        return _
    lax.fori_loop(0, nrows, _zero_row, None)


def _sc_a_body(xl, xr, nidx, eidx, de, hxl, hxr,
               hx_sh, zb, sbuf, rows, nv, ev, dev, sem):
    c = lax.axis_index("c")
    t = lax.axis_index("s")

    _zero_vmem(zb, ZR)
    for k in range(E_T // ZR):
        pltpu.sync_copy(zb, hx_sh.at[pl.ds(t * E_T + k * ZR, ZR)])
    pltpu.sync_copy(de.at[pl.ds(t * E_T, E_T)], dev)
    plsc.subcore_barrier()

    def _p1(i, _):
        base = t * NNZ_T + i * C
        pltpu.sync_copy(nidx.at[pl.ds(base, C)], nv)
        pltpu.sync_copy(eidx.at[pl.ds(base, C)], ev)

        @pl.when(c == 0)
        def _():
            pltpu.async_copy(xl.at[nv], rows, sem).wait()

        @pl.when(c == 1)
        def _():
            pltpu.async_copy(xr.at[nv], rows, sem).wait()

        pltpu.sync_copy(rows, hx_sh.at[ev], add=True)
        return _
    lax.fori_loop(0, N_ITER, _p1, None)
    plsc.subcore_barrier()

    def _scale16(g, _):
        pltpu.sync_copy(hx_sh.at[pl.ds(t * E_T + g * 16, 16)], sbuf)
        d16 = dev[pl.ds(g * 16, 16)]
        for j in range(16):
            dj = jnp.broadcast_to(d16[j], (16,))
            for v in range(HD // 16):
                sbuf[j, pl.ds(v * 16, 16)] = sbuf[j, pl.ds(v * 16, 16)] * dj

        @pl.when(c == 0)
        def _():
            pltpu.sync_copy(sbuf, hxl.at[pl.ds(t * E_T + g * 16, 16)])

        @pl.when(c == 1)
        def _():
            pltpu.sync_copy(sbuf, hxr.at[pl.ds(t * E_T + g * 16, 16)])
        return _
    lax.fori_loop(0, E_T // 16, _scale16, None)


def _sc_b_body(hxl, hxr, nidx, eidx, outl, outr,
               xo_sh, zb, rows, nv, ev, sem):
    c = lax.axis_index("c")
    t = lax.axis_index("s")

    _zero_vmem(zb, ZR)
    for k in range(NODE_T // ZR):
        pltpu.sync_copy(zb, xo_sh.at[pl.ds(t * NODE_T + k * ZR, ZR)])
    rem = NODE_T % ZR
    pltpu.sync_copy(zb.at[pl.ds(0, rem)],
                    xo_sh.at[pl.ds(t * NODE_T + NODE_T - rem, rem)])
    plsc.subcore_barrier()

    def _p2(i, _):
        base = t * NNZ_T + i * C
        pltpu.sync_copy(nidx.at[pl.ds(base, C)], nv)
        pltpu.sync_copy(eidx.at[pl.ds(base, C)], ev)

        @pl.when(c == 0)
        def _():
            pltpu.async_copy(hxl.at[ev], rows, sem).wait()

        @pl.when(c == 1)
        def _():
            pltpu.async_copy(hxr.at[ev], rows, sem).wait()

        pltpu.sync_copy(rows, xo_sh.at[nv], add=True)
        return _
    lax.fori_loop(0, N_ITER, _p2, None)
    plsc.subcore_barrier()

    def _wout(k, _):
        pltpu.sync_copy(xo_sh.at[pl.ds(t * NODE_T + k * ZR, ZR)], zb)

        @pl.when(c == 0)
        def _():
            pltpu.sync_copy(zb, outl.at[pl.ds(t * NODE_T + k * ZR, ZR)])

        @pl.when(c == 1)
        def _():
            pltpu.sync_copy(zb, outr.at[pl.ds(t * NODE_T + k * ZR, ZR)])
        return _
    lax.fori_loop(0, NODE_T // ZR, _wout, None)

    pltpu.sync_copy(xo_sh.at[pl.ds(t * NODE_T + NODE_T - rem, rem)],
                    zb.at[pl.ds(0, rem)])

    @pl.when(c == 0)
    def _():
        pltpu.sync_copy(zb.at[pl.ds(0, rem)],
                        outl.at[pl.ds(t * NODE_T + NODE_T - rem, rem)])

    @pl.when(c == 1)
    def _():
        pltpu.sync_copy(zb.at[pl.ds(0, rem)],
                        outr.at[pl.ds(t * NODE_T + NODE_T - rem, rem)])


_sc_a = pl.kernel(
    _sc_a_body,
    out_type=[jax.ShapeDtypeStruct((EHP, HD), jnp.float32),
              jax.ShapeDtypeStruct((EHP, HD), jnp.float32)],
    mesh=plsc.VectorSubcoreMesh(core_axis_name="c", subcore_axis_name="s"),
    scratch_types=[
        pltpu.VMEM_SHARED((EHP, HD), jnp.float32),
        pltpu.VMEM((ZR, HD), jnp.float32),
        pltpu.VMEM((16, HD), jnp.float32),
        pltpu.VMEM((C, HD), jnp.float32),
        pltpu.VMEM((C,), jnp.int32),
        pltpu.VMEM((C,), jnp.int32),
        pltpu.VMEM((E_T,), jnp.float32),
        pltpu.SemaphoreType.DMA,
    ],
)

_sc_b = pl.kernel(
    _sc_b_body,
    out_type=[jax.ShapeDtypeStruct((NP, HD), jnp.float32),
              jax.ShapeDtypeStruct((NP, HD), jnp.float32)],
    mesh=plsc.VectorSubcoreMesh(core_axis_name="c", subcore_axis_name="s"),
    scratch_types=[
        pltpu.VMEM_SHARED((NP, HD), jnp.float32),
        pltpu.VMEM((ZR, HD), jnp.float32),
        pltpu.VMEM((C, HD), jnp.float32),
        pltpu.VMEM((C,), jnp.int32),
        pltpu.VMEM((C,), jnp.int32),
        pltpu.SemaphoreType.DMA,
    ],
)


def _pre_body(x_ref, dv_ref, l_ref, r_ref):
    xn = x_ref[...] * dv_ref[...]
    l_ref[...] = xn[:, :HD]
    r_ref[...] = xn[:, HD:]


_tc_pre = pl.pallas_call(
    _pre_body,
    grid=(N // BN,),
    in_specs=[pl.BlockSpec((BN, D), lambda i: (i, 0)),
              pl.BlockSpec((BN, 1), lambda i: (i, 0))],
    out_specs=[pl.BlockSpec((BN, HD), lambda i: (i, 0)),
               pl.BlockSpec((BN, HD), lambda i: (i, 0))],
    out_shape=[jax.ShapeDtypeStruct((N, HD), jnp.float32),
               jax.ShapeDtypeStruct((N, HD), jnp.float32)],
)


def _post_body(l_ref, r_ref, dv_ref, res_ref, w_ref, b_ref, g_ref, be_ref,
               xh_ref, xnl_ref, xnr_ref):
    dv = dv_ref[...]
    xo = jnp.concatenate([l_ref[...], r_ref[...]], axis=1) * dv
    y = jnp.dot(xo, w_ref[...], preferred_element_type=jnp.float32)
    y = y + b_ref[...] + res_ref[...]
    mu = jnp.mean(y, axis=1, keepdims=True)
    yc = y - mu
    var = jnp.mean(yc * yc, axis=1, keepdims=True)
    z = yc * lax.rsqrt(var + 1e-5) * g_ref[...] + be_ref[...]
    xh = jnp.maximum(z, 0.0)
    xh_ref[...] = xh
    xn = xh * dv
    xnl_ref[...] = xn[:, :HD]
    xnr_ref[...] = xn[:, HD:]


_tc_post = pl.pallas_call(
    _post_body,
    grid=(N // BN,),
    in_specs=[pl.BlockSpec((BN, HD), lambda i: (i, 0)),
              pl.BlockSpec((BN, HD), lambda i: (i, 0)),
              pl.BlockSpec((BN, 1), lambda i: (i, 0)),
              pl.BlockSpec((BN, D), lambda i: (i, 0)),
              pl.BlockSpec((D, D), lambda i: (0, 0)),
              pl.BlockSpec((1, D), lambda i: (0, 0)),
              pl.BlockSpec((1, D), lambda i: (0, 0)),
              pl.BlockSpec((1, D), lambda i: (0, 0))],
    out_specs=[pl.BlockSpec((BN, D), lambda i: (i, 0)),
               pl.BlockSpec((BN, HD), lambda i: (i, 0)),
               pl.BlockSpec((BN, HD), lambda i: (i, 0))],
    out_shape=[jax.ShapeDtypeStruct((N, D), jnp.float32),
               jax.ShapeDtypeStruct((N, HD), jnp.float32),
               jax.ShapeDtypeStruct((N, HD), jnp.float32)],
)


def _fuse_body(x1_ref, x2_ref, wa_ref, ba_ref, o_ref):
    a = x1_ref[...]
    b = x2_ref[...]
    wv = wa_ref[...]
    s1 = jnp.dot(a, wv, preferred_element_type=jnp.float32) + ba_ref[...]
    s2 = jnp.dot(b, wv, preferred_element_type=jnp.float32) + ba_ref[...]
    m = jnp.maximum(s1, s2)
    e1 = jnp.exp(s1 - m)
    e2 = jnp.exp(s2 - m)
    w1 = e1 / (e1 + e2)
    o_ref[...] = w1 * a + (1.0 - w1) * b


_tc_fuse = pl.pallas_call(
    _fuse_body,
    grid=(N // BN,),
    in_specs=[pl.BlockSpec((BN, D), lambda i: (i, 0)),
              pl.BlockSpec((BN, D), lambda i: (i, 0)),
              pl.BlockSpec((D, 1), lambda i: (0, 0)),
              pl.BlockSpec((1, 1), lambda i: (0, 0))],
    out_specs=pl.BlockSpec((BN, D), lambda i: (i, 0)),
    out_shape=jax.ShapeDtypeStruct((N, D), jnp.float32),
)


def kernel(X, h1_node_idx, h1_edge_idx, h1_DV_inv_sqrt, h1_DE_inv,
           h2_node_idx, h2_edge_idx, h2_DV_inv_sqrt, h2_DE_inv,
           W1, b1, W2, b2, g1, beta1, g2, beta2, Wa, ba):
    params = [(W1, b1.reshape(1, D), g1.reshape(1, D), beta1.reshape(1, D)),
              (W2, b2.reshape(1, D), g2.reshape(1, D), beta2.reshape(1, D))]

    def branch(nidx, eidx, dv, de):
        nidx = nidx.astype(jnp.int32)
        eidx = eidx.astype(jnp.int32)
        dv2 = dv.reshape(N, 1)
        dep = jnp.pad(de, (0, EHP - EH))
        xh = X
        xnl, xnr = _tc_pre(X, dv2)
        for w, bb, gg, be in params:
            hxl, hxr = _sc_a(xnl, xnr, nidx, eidx, dep)
            ol, orr = _sc_b(hxl, hxr, nidx, eidx)
            xh, xnl, xnr = _tc_post(ol, orr, dv2, xh, w, bb, gg, be)
        return xh

    X1 = branch(h1_node_idx, h1_edge_idx, h1_DV_inv_sqrt, h1_DE_inv)
    X2 = branch(h2_node_idx, h2_edge_idx, h2_DV_inv_sqrt, h2_DE_inv)
    return _tc_fuse(X1, X2, Wa, ba.reshape(1, 1))
